# baseline (speedup 1.0000x reference)
import functools

import jax
import jax.numpy as jnp
from jax import lax
from jax.experimental import pallas as pl
from jax.experimental.pallas import tpu as pltpu
from jax.experimental.pallas import tpu_sc as plsc

HDIM = 64
B = 1024
L = 200
NPAIR = B * L
NC = 2
NS = 16
NW = NC * NS
PPW = NPAIR // NW
CK = 128
NCH = PPW // CK
CPW = B // NW


def _sc_gather_body(nemb, nfeat, efeat,
                    ps, po, pe, ns_, no_, ne_, cs, co,
                    out_pos, out_neg, out_c,
                    idx_s, idx_o, idx_e, acc, acc_c, sem):
  wid = lax.axis_index("s") * NC + lax.axis_index("c")

  def do_side(sids_hbm, oids_hbm, eids_hbm, out_hbm):
    def chunk(j, carry):
      base = wid * PPW + j * CK
      pltpu.sync_copy(sids_hbm.at[pl.ds(base, CK)], idx_s)
      pltpu.sync_copy(oids_hbm.at[pl.ds(base, CK)], idx_o)
      pltpu.sync_copy(eids_hbm.at[pl.ds(base, CK)], idx_e)
      pltpu.async_copy(nemb.at[idx_s], acc, sem).wait()
      c1 = pltpu.async_copy(nemb.at[idx_o], acc, sem, add=True)
      c2 = pltpu.async_copy(nfeat.at[idx_s], acc, sem, add=True)
      c3 = pltpu.async_copy(nfeat.at[idx_o], acc, sem, add=True)
      c4 = pltpu.async_copy(efeat.at[idx_e], acc, sem, add=True)
      c1.wait(); c2.wait(); c3.wait(); c4.wait()
      pltpu.sync_copy(acc, out_hbm.at[pl.ds(base, CK)])
      return carry
    lax.fori_loop(0, NCH, chunk, 0)

  do_side(ps, po, pe, out_pos)
  do_side(ns_, no_, ne_, out_neg)

  cbase = wid * CPW
  idx_cs = idx_s.at[pl.ds(0, CPW)]
  idx_co = idx_o.at[pl.ds(0, CPW)]
  pltpu.sync_copy(cs.at[pl.ds(cbase, CPW)], idx_cs)
  pltpu.sync_copy(co.at[pl.ds(cbase, CPW)], idx_co)
  pltpu.async_copy(nemb.at[idx_cs], acc_c, sem).wait()
  c1 = pltpu.async_copy(nemb.at[idx_co], acc_c, sem, add=True)
  c2 = pltpu.async_copy(nfeat.at[idx_cs], acc_c, sem, add=True)
  c3 = pltpu.async_copy(nfeat.at[idx_co], acc_c, sem, add=True)
  c1.wait(); c2.wait(); c3.wait()
  pltpu.sync_copy(acc_c, out_c.at[pl.ds(cbase, CPW)])


_sc_gather = pl.kernel(
    _sc_gather_body,
    out_type=(
        jax.ShapeDtypeStruct((NPAIR, HDIM), jnp.float32),
        jax.ShapeDtypeStruct((NPAIR, HDIM), jnp.float32),
        jax.ShapeDtypeStruct((B, HDIM), jnp.float32),
    ),
    mesh=plsc.VectorSubcoreMesh(core_axis_name="c", subcore_axis_name="s"),
    scratch_types=[
        pltpu.VMEM((CK,), jnp.int32),
        pltpu.VMEM((CK,), jnp.int32),
        pltpu.VMEM((CK,), jnp.int32),
        pltpu.VMEM((CK, HDIM), jnp.float32),
        pltpu.VMEM((CPW, HDIM), jnp.float32),
        pltpu.SemaphoreType.DMA,
    ],
    compiler_params=pltpu.CompilerParams(use_tc_tiling_on_sc=False),
)


BB = 8


def _tc_body(embp_ref, embn_ref, embc_ref, cts_ref, pts_ref, nts_ref,
             bf_ref, ph_ref, pos_out, neg_out):
  bf = bf_ref[0, :]
  ph = ph_ref[0, :]
  cemb = embc_ref[...] + jnp.cos(ph)[None, :]
  cts = cts_ref[...]

  def side(emb_ref, ts_ref, out_ref):
    td = cts - ts_ref[...]
    ang = td[:, :, None] * bf[None, None, :] + ph[None, None, :]
    e = emb_ref[...] + jnp.cos(ang)
    out_ref[...] = jnp.sum(e * cemb[:, None, :], axis=-1)

  side(embp_ref, pts_ref, pos_out)
  side(embn_ref, nts_ref, neg_out)


_tc_call = pl.pallas_call(
    _tc_body,
    grid=(B // BB,),
    in_specs=[
        pl.BlockSpec((BB, L, HDIM), lambda i: (i, 0, 0)),
        pl.BlockSpec((BB, L, HDIM), lambda i: (i, 0, 0)),
        pl.BlockSpec((BB, HDIM), lambda i: (i, 0)),
        pl.BlockSpec((BB, 1), lambda i: (i, 0)),
        pl.BlockSpec((BB, L), lambda i: (i, 0)),
        pl.BlockSpec((BB, L), lambda i: (i, 0)),
        pl.BlockSpec((1, HDIM), lambda i: (0, 0)),
        pl.BlockSpec((1, HDIM), lambda i: (0, 0)),
    ],
    out_specs=[
        pl.BlockSpec((BB, L), lambda i: (i, 0)),
        pl.BlockSpec((BB, L), lambda i: (i, 0)),
    ],
    out_shape=[
        jax.ShapeDtypeStruct((B, L), jnp.float32),
        jax.ShapeDtypeStruct((B, L), jnp.float32),
    ],
)


@jax.jit
def kernel(c_sids, c_oids, c_eids, c_ts,
           pos_sids, pos_oids, pos_eids, pos_ts,
           neg_sids, neg_oids, neg_eids, neg_ts,
           nemb, nfeat, efeat, basis_freq, phase):
  i32 = jnp.int32
  ps = pos_sids.astype(i32).reshape(NPAIR)
  po = pos_oids.astype(i32).reshape(NPAIR)
  pe = pos_eids.astype(i32).reshape(NPAIR)
  ns_ = neg_sids.astype(i32).reshape(NPAIR)
  no_ = neg_oids.astype(i32).reshape(NPAIR)
  ne_ = neg_eids.astype(i32).reshape(NPAIR)
  cs = c_sids.astype(i32).reshape(B)
  co = c_oids.astype(i32).reshape(B)

  emb_pos, emb_neg, emb_c = _sc_gather(
      nemb, nfeat, efeat, ps, po, pe, ns_, no_, ne_, cs, co)

  pos_dot, neg_dot = _tc_call(
      emb_pos.reshape(B, L, HDIM),
      emb_neg.reshape(B, L, HDIM),
      emb_c,
      c_ts,
      pos_ts,
      neg_ts,
      basis_freq.reshape(1, HDIM),
      phase.reshape(1, HDIM),
  )
  return (pos_dot, neg_dot)
